# baseline (speedup 1.0000x reference)
import jax
import jax.numpy as jnp
from jax import lax
from jax.experimental import pallas as pl
from jax.experimental.pallas import tpu as pltpu
from jax.experimental.pallas import tpu_sc as plsc

_NC, _NS = 2, 16
_NW = _NC * _NS
_V = 1000000
_D = 64
_NFULL = _V // 128
_VMAIN = _NFULL * 128
_NTAIL = _V - _VMAIN


def _bcast(i16, scalar):
    return i16 * 0 + scalar


def _shuffle_slab(adr, in_slab, out1d):
    @plsc.parallel_loop(0, 64, unroll=8)
    def row(d):
        for g in range(8):
            a = adr[pl.ds(d * 128 + g * 16, 16)]
            x = in_slab[d, pl.ds(g * 16, 16)]
            plsc.store_scatter(out1d, [a], x)


def _fill_adr(i16, adr):
    @plsc.parallel_loop(0, 64, unroll=8)
    def row(d):
        for g in range(8):
            adr[pl.ds(d * 128 + g * 16, 16)] = i16 * 64 + (1024 * g + d)


def _tr_body(wt, tail, tp, adr, in0, in1, o0, o1, rs0, rs1, ws0, ws1):
    wid = lax.axis_index("s") * _NC + lax.axis_index("c")
    i16 = lax.iota(jnp.int32, 16)
    ins, outs, rsems, wsems = (in0, in1), (o0, o1), (rs0, rs1), (ws0, ws1)
    _fill_adr(i16, adr)

    def chunk_of(k):
        return wid + _NW * k

    def start_read(k, b):
        c = chunk_of(k)
        pltpu.async_copy(wt.at[:, pl.ds(c * 128, 128)], ins[b], rsems[b])

    def wait_read(b):
        pltpu.make_async_copy(wt.at[:, pl.ds(0, 128)], ins[b], rsems[b]).wait()

    def start_write(k, b):
        c = chunk_of(k)
        pltpu.async_copy(outs[b], tp.at[pl.ds(c * 8192, 8192)], wsems[b])

    def wait_write(b):
        pltpu.make_async_copy(tp.at[pl.ds(0, 8192)], outs[b], wsems[b]).wait()

    start_read(0, 0)
    start_read(1, 1)
    for b in range(2):
        wait_read(b)
        _shuffle_slab(adr, ins[b], outs[b])
        start_write(b, b)
        start_read(b + 2, b)

    def step(k2, carry):
        for b in range(2):
            k = 2 * k2 + b
            wait_read(b)
            wait_write(b)
            _shuffle_slab(adr, ins[b], outs[b])
            start_write(k, b)
            start_read(k + 2, b)
        return carry

    lax.fori_loop(1, 121, step, 0)

    for b in range(2):
        wait_read(b)
        wait_write(b)
        _shuffle_slab(adr, ins[b], outs[b])
        start_write(242 + b, b)
    for b in range(2):
        wait_write(b)

    @pl.when(wid < 4)
    def _():
        c = 7808 + wid
        pltpu.sync_copy(wt.at[:, pl.ds(c * 128, 128)], in0)
        _shuffle_slab(adr, in0, o0)
        pltpu.sync_copy(o0, tp.at[pl.ds(c * 8192, 8192)])

    @pl.when(wid == 4)
    def _():
        pltpu.sync_copy(tail, o1.at[pl.ds(0, 4096)])
        pltpu.sync_copy(o1.at[pl.ds(0, 4096)], tp.at[pl.ds(_VMAIN * 64, 4096)])


def _pair_tail(embed_weight):
    return embed_weight[_VMAIN:, :].reshape(_NTAIL * _D)


def _gb_body(tp, idx, out, idxf, ir0, ir1, sb0, sb1, pb0, pb1, ob0, ob1,
             gs0, gs1, ws0, ws1):
    wid = lax.axis_index("s") * _NC + lax.axis_index("c")
    i16 = lax.iota(jnp.int32, 16)
    irs, sbs, pbs, obs = (ir0, ir1), (sb0, sb1), (pb0, pb1), (ob0, ob1)
    gsems, wsems = (gs0, gs1), (ws0, ws1)

    pltpu.sync_copy(idx.at[pl.ds(wid * 6400, 6400)], idxf)
    pat50 = i16 * 50

    def stage_idx(h, b):
        for g in range(8):
            addr = pat50 + (g * 16 * 50 + h)
            v = plsc.load_gather(idxf, [addr])
            irs[b][pl.ds(g * 16, 16)] = v >> 1
            sbs[b][pl.ds(g * 16, 16)] = (v & 1) * 64
        pltpu.async_copy(tp.at[irs[b]], pbs[b].at[:, pl.ds(0, 128)], gsems[b])

    def wait_gather(b):
        pltpu.make_async_copy(
            tp.at[pl.ds(0, 128), :], pbs[b].at[:, pl.ds(0, 128)], gsems[b]
        ).wait()

    def shuffle_out(b):
        @plsc.parallel_loop(0, 64, unroll=8)
        def row(d):
            for g in range(8):
                col = sbs[b][pl.ds(g * 16, 16)] + d
                x = plsc.load_gather(pbs[b], [i16 + g * 16, col])
                obs[b][d, pl.ds(g * 16, 16)] = x

    def start_write(h, b):
        pltpu.async_copy(obs[b], out.at[h, :, pl.ds(wid * 128, 128)], wsems[b])

    def wait_write(b):
        pltpu.make_async_copy(tp.at[pl.ds(0, 64), :], obs[b], wsems[b]).wait()

    stage_idx(0, 0)
    stage_idx(1, 1)
    for b in range(2):
        wait_gather(b)
        shuffle_out(b)
        start_write(b, b)
        stage_idx(b + 2, b)

    def step(h2, carry):
        for b in range(2):
            h = 2 * h2 + b
            wait_gather(b)
            wait_write(b)
            shuffle_out(b)
            start_write(h, b)
            stage_idx(h + 2, b)
        return carry

    lax.fori_loop(1, 24, step, 0)

    for b in range(2):
        wait_gather(b)
        wait_write(b)
        shuffle_out(b)
        start_write(48 + b, b)
    for b in range(2):
        wait_write(b)


def kernel(x, embed_weight):
    B, H = x.shape
    V, D = embed_weight.shape
    mesh = plsc.VectorSubcoreMesh(core_axis_name="c", subcore_axis_name="s")
    params = pltpu.CompilerParams(
        use_tc_tiling_on_sc=True, needs_layout_passes=False
    )

    wt = embed_weight.T
    tail = _pair_tail(embed_weight)

    transpose_fn = pl.kernel(
        _tr_body,
        out_type=jax.ShapeDtypeStruct((V * D,), jnp.float32),
        mesh=mesh,
        compiler_params=params,
        scratch_types=[
            pltpu.VMEM((8192,), jnp.int32),
            pltpu.VMEM((64, 128), jnp.float32),
            pltpu.VMEM((64, 128), jnp.float32),
            pltpu.VMEM((8192,), jnp.float32),
            pltpu.VMEM((8192,), jnp.float32),
            pltpu.SemaphoreType.DMA,
            pltpu.SemaphoreType.DMA,
            pltpu.SemaphoreType.DMA,
            pltpu.SemaphoreType.DMA,
        ],
    )
    tp = transpose_fn(wt, tail).reshape(V // 2, 128)

    idx = x.reshape(B * H).astype(jnp.int32)
    gather_fn = pl.kernel(
        _gb_body,
        out_type=jax.ShapeDtypeStruct((H, D, B), jnp.float32),
        mesh=mesh,
        compiler_params=params,
        scratch_types=[
            pltpu.VMEM((6400,), jnp.int32),
            pltpu.VMEM((128,), jnp.int32),
            pltpu.VMEM((128,), jnp.int32),
            pltpu.VMEM((128,), jnp.int32),
            pltpu.VMEM((128,), jnp.int32),
            pltpu.VMEM((128, 137), jnp.float32),
            pltpu.VMEM((128, 137), jnp.float32),
            pltpu.VMEM((64, 128), jnp.float32),
            pltpu.VMEM((64, 128), jnp.float32),
            pltpu.SemaphoreType.DMA,
            pltpu.SemaphoreType.DMA,
            pltpu.SemaphoreType.DMA,
            pltpu.SemaphoreType.DMA,
        ],
    )
    outb = gather_fn(tp, idx)
    return jnp.transpose(outb, (2, 0, 1))


# speedup vs baseline: 1.6181x; 1.6181x over previous
import jax
import jax.numpy as jnp
from jax import lax
from jax.experimental import pallas as pl
from jax.experimental.pallas import tpu as pltpu
from jax.experimental.pallas import tpu_sc as plsc

_NC, _NS = 2, 16
_NW = _NC * _NS


def _bcast(i16, scalar):
    return i16 * 0 + scalar


def _gb_body(tp, idx, out, idxf, ir0, ir1, pb0, pb1, ob0, ob1,
             gs0, gs1, ws0, ws1):
    wid = lax.axis_index("s") * _NC + lax.axis_index("c")
    i16 = lax.iota(jnp.int32, 16)
    irs, pbs, obs = (ir0, ir1), (pb0, pb1), (ob0, ob1)
    gsems, wsems = (gs0, gs1), (ws0, ws1)

    pltpu.sync_copy(idx.at[pl.ds(wid * 6400, 6400)], idxf)
    pat50 = i16 * 50

    def stage_idx(h, b):
        for g in range(8):
            addr = pat50 + (g * 16 * 50 + h)
            irs[b][pl.ds(g * 16, 16)] = plsc.load_gather(idxf, [addr])
        pltpu.async_copy(tp.at[irs[b]], pbs[b], gsems[b])

    def wait_gather(b):
        pltpu.make_async_copy(tp.at[pl.ds(0, 128), :], pbs[b], gsems[b]).wait()

    def shuffle_out(b):
        @plsc.parallel_loop(0, 64, unroll=8)
        def row(d):
            col = _bcast(i16, d)
            for g in range(8):
                x = plsc.load_gather(pbs[b], [i16 + g * 16, col])
                obs[b][d, pl.ds(g * 16, 16)] = x

    def start_write(h, b):
        pltpu.async_copy(obs[b], out.at[h, :, pl.ds(wid * 128, 128)], wsems[b])

    def wait_write(b):
        pltpu.make_async_copy(tp.at[pl.ds(0, 64), :], obs[b], wsems[b]).wait()

    stage_idx(0, 0)
    stage_idx(1, 1)
    for b in range(2):
        wait_gather(b)
        shuffle_out(b)
        start_write(b, b)
        stage_idx(b + 2, b)

    def step(h2, carry):
        for b in range(2):
            h = 2 * h2 + b
            wait_gather(b)
            wait_write(b)
            shuffle_out(b)
            start_write(h, b)
            stage_idx(h + 2, b)
        return carry

    lax.fori_loop(1, 24, step, 0)

    for b in range(2):
        wait_gather(b)
        wait_write(b)
        shuffle_out(b)
        start_write(48 + b, b)
    for b in range(2):
        wait_write(b)


def kernel(x, embed_weight):
    B, H = x.shape
    V, D = embed_weight.shape
    mesh = plsc.VectorSubcoreMesh(core_axis_name="c", subcore_axis_name="s")
    params = pltpu.CompilerParams(
        use_tc_tiling_on_sc=True, needs_layout_passes=False
    )

    tp = jnp.pad(embed_weight, ((0, 0), (0, 128 - D)))
    idx = x.reshape(B * H).astype(jnp.int32)

    gather_fn = pl.kernel(
        _gb_body,
        out_type=jax.ShapeDtypeStruct((H, D, B), jnp.float32),
        mesh=mesh,
        compiler_params=params,
        scratch_types=[
            pltpu.VMEM((6400,), jnp.int32),
            pltpu.VMEM((128,), jnp.int32),
            pltpu.VMEM((128,), jnp.int32),
            pltpu.VMEM((128, 128), jnp.float32),
            pltpu.VMEM((128, 128), jnp.float32),
            pltpu.VMEM((64, 128), jnp.float32),
            pltpu.VMEM((64, 128), jnp.float32),
            pltpu.SemaphoreType.DMA,
            pltpu.SemaphoreType.DMA,
            pltpu.SemaphoreType.DMA,
            pltpu.SemaphoreType.DMA,
        ],
    )
    outb = gather_fn(tp, idx)
    return jnp.transpose(outb, (2, 0, 1))


# speedup vs baseline: 1.6185x; 1.0003x over previous
import jax
import jax.numpy as jnp
from jax import lax
from jax.experimental import pallas as pl
from jax.experimental.pallas import tpu as pltpu
from jax.experimental.pallas import tpu_sc as plsc

_NC, _NS = 2, 16
_NW = _NC * _NS


def _bcast(i16, scalar):
    return i16 * 0 + scalar


def _gb_body(tp, idx, out, idxf, ir0, ir1, pb0, pb1, ob0, ob1,
             gs0, gs1, ws0, ws1):
    wid = lax.axis_index("s") * _NC + lax.axis_index("c")
    i16 = lax.iota(jnp.int32, 16)
    irs, pbs, obs = (ir0, ir1), (pb0, pb1), (ob0, ob1)
    gsems, wsems = (gs0, gs1), (ws0, ws1)

    pltpu.sync_copy(idx.at[pl.ds(wid * 6400, 6400)], idxf)
    pat50 = i16 * 50

    def stage_idx(h, b):
        for g in range(8):
            addr = pat50 + (g * 16 * 50 + h)
            irs[b][pl.ds(g * 16, 16)] = plsc.load_gather(idxf, [addr])
        pltpu.async_copy(tp.at[irs[b]], pbs[b], gsems[b])

    def wait_gather(b):
        pltpu.make_async_copy(tp.at[pl.ds(0, 128), :], pbs[b], gsems[b]).wait()

    def shuffle_out(b):
        @plsc.parallel_loop(0, 64, unroll=8)
        def row(d):
            col = _bcast(i16, d)
            for g in range(8):
                x = plsc.load_gather(pbs[b], [i16 + g * 16, col])
                obs[b][d, pl.ds(g * 16, 16)] = x

    def start_write(h, b):
        pltpu.async_copy(obs[b], out.at[h, :, pl.ds(wid * 128, 128)], wsems[b])

    def wait_write(b):
        pltpu.make_async_copy(tp.at[pl.ds(0, 64), :], obs[b], wsems[b]).wait()

    stage_idx(0, 0)
    stage_idx(1, 1)
    for b in range(2):
        wait_gather(b)
        shuffle_out(b)
        start_write(b, b)
        stage_idx(b + 2, b)

    def step(h2, carry):
        for b in range(2):
            h = 2 * h2 + b
            wait_gather(b)
            wait_write(b)
            shuffle_out(b)
            start_write(h, b)
            stage_idx(h + 2, b)
        return carry

    lax.fori_loop(1, 24, step, 0)

    for b in range(2):
        wait_gather(b)
        wait_write(b)
        shuffle_out(b)
        start_write(48 + b, b)
    for b in range(2):
        wait_write(b)


def kernel(x, embed_weight):
    B, H = x.shape
    V, D = embed_weight.shape
    mesh = plsc.VectorSubcoreMesh(core_axis_name="c", subcore_axis_name="s")
    params = pltpu.CompilerParams(
        use_tc_tiling_on_sc=True, needs_layout_passes=False
    )

    tp = jnp.concatenate(
        [embed_weight, jnp.zeros((V, 128 - D), jnp.float32)], axis=1
    )
    idx = x.reshape(B * H).astype(jnp.int32)

    gather_fn = pl.kernel(
        _gb_body,
        out_type=jax.ShapeDtypeStruct((H, D, B), jnp.float32),
        mesh=mesh,
        compiler_params=params,
        scratch_types=[
            pltpu.VMEM((6400,), jnp.int32),
            pltpu.VMEM((128,), jnp.int32),
            pltpu.VMEM((128,), jnp.int32),
            pltpu.VMEM((128, 128), jnp.float32),
            pltpu.VMEM((128, 128), jnp.float32),
            pltpu.VMEM((64, 128), jnp.float32),
            pltpu.VMEM((64, 128), jnp.float32),
            pltpu.SemaphoreType.DMA,
            pltpu.SemaphoreType.DMA,
            pltpu.SemaphoreType.DMA,
            pltpu.SemaphoreType.DMA,
        ],
    )
    outb = gather_fn(tp, idx)
    return jnp.transpose(outb, (2, 0, 1))
